# Initial kernel scaffold; baseline (speedup 1.0000x reference)
import jax
import jax.numpy as jnp
from jax.experimental import pallas as pl


def kernel(node_attr, edge_index, edge_attr, W1, b1, W2, b2, root):
    raise NotImplementedError("write your pallas kernel here")


# speedup vs baseline: 1.5673x; 1.5673x over previous
import functools

import jax
import jax.numpy as jnp
from jax import lax
from jax.experimental import pallas as pl
from jax.experimental.pallas import tpu as pltpu
from jax.experimental.pallas import tpu_sc as plsc

_N = 10000
_E = 160000
_DIN = 128
_DOUT = 8
_DE = 16
_HID = 128

_CHUNK = 128
_NCH = _E // _CHUNK
_NC = 2
_NS = 16
_NW = _NC * _NS
_CPW = -(-_NCH // _NW)
_NPW = _N // _NS

_sc_mesh = plsc.VectorSubcoreMesh(core_axis_name="c", subcore_axis_name="s")


@functools.partial(
    pl.kernel,
    mesh=_sc_mesh,
    out_type=jax.ShapeDtypeStruct((_E, _DIN), jnp.float32),
    scratch_types=[
        pltpu.VMEM((_CHUNK,), jnp.int32),
        pltpu.VMEM((_CHUNK, _DIN), jnp.float32),
        pltpu.SemaphoreType.DMA,
    ],
    compiler_params=pltpu.CompilerParams(use_tc_tiling_on_sc=False),
)
def _sc_gather(na_hbm, src_hbm, out_hbm, idx_v, rows_v, sem):
    wid = lax.axis_index("s") * _NC + lax.axis_index("c")

    def body(j, carry):
        c = j * _NW + wid

        @pl.when(c < _NCH)
        def _():
            base = pl.multiple_of(c * _CHUNK, _CHUNK)
            pltpu.sync_copy(src_hbm.at[pl.ds(base, _CHUNK)], idx_v)
            pltpu.async_copy(na_hbm.at[idx_v], rows_v, sem).wait()
            pltpu.sync_copy(rows_v, out_hbm.at[pl.ds(base, _CHUNK)])

        return carry

    lax.fori_loop(0, _CPW, body, 0)


_BE = 2000


def _msg_body(ea_ref, x_ref, w1_ref, b1_ref, wc_ref, b2r_ref, o_ref):
    x = x_ref[...]
    h = jnp.maximum(
        jnp.dot(ea_ref[...], w1_ref[...], preferred_element_type=jnp.float32)
        + b1_ref[...],
        0.0,
    )
    q = jnp.dot(x, wc_ref[...], preferred_element_type=jnp.float32)
    t = q.reshape(_BE, _DOUT, _HID) * h[:, None, :]
    m = jnp.sum(t, axis=-1)
    o_ref[...] = m + jnp.dot(x, b2r_ref[...], preferred_element_type=jnp.float32)


_msg_call = pl.pallas_call(
    _msg_body,
    grid=(_E // _BE,),
    in_specs=[
        pl.BlockSpec((_BE, _DE), lambda i: (i, 0)),
        pl.BlockSpec((_BE, _DIN), lambda i: (i, 0)),
        pl.BlockSpec((_DE, _HID), lambda i: (0, 0)),
        pl.BlockSpec((1, _HID), lambda i: (0, 0)),
        pl.BlockSpec((_DIN, _DOUT * _HID), lambda i: (0, 0)),
        pl.BlockSpec((_DIN, _DOUT), lambda i: (0, 0)),
    ],
    out_specs=pl.BlockSpec((_BE, _DOUT), lambda i: (i, 0)),
    out_shape=jax.ShapeDtypeStruct((_E, _DOUT), jnp.float32),
    compiler_params=pltpu.CompilerParams(
        dimension_semantics=("arbitrary",),
    ),
)


@functools.partial(
    pl.kernel,
    mesh=_sc_mesh,
    out_type=jax.ShapeDtypeStruct((_NC * _N, _DOUT), jnp.float32),
    scratch_types=[
        pltpu.VMEM((_CHUNK,), jnp.int32),
        pltpu.VMEM((_CHUNK, _DOUT), jnp.float32),
        pltpu.VMEM_SHARED((_N, _DOUT), jnp.float32),
    ],
    compiler_params=pltpu.CompilerParams(use_tc_tiling_on_sc=False),
)
def _sc_scatter(msg_hbm, dst_hbm, zero_hbm, out_hbm, idx_v, msg_v, acc_sh):
    cid = lax.axis_index("c")
    sid = lax.axis_index("s")
    wid = sid * _NC + cid

    stripe = pl.ds(sid * _NPW, _NPW)
    pltpu.sync_copy(zero_hbm.at[stripe], acc_sh.at[stripe])
    plsc.subcore_barrier()

    def body(j, carry):
        c = j * _NW + wid

        @pl.when(c < _NCH)
        def _():
            base = pl.multiple_of(c * _CHUNK, _CHUNK)
            pltpu.sync_copy(dst_hbm.at[pl.ds(base, _CHUNK)], idx_v)
            pltpu.sync_copy(msg_hbm.at[pl.ds(base, _CHUNK)], msg_v)
            pltpu.sync_copy(msg_v, acc_sh.at[idx_v], add=True)

        return carry

    lax.fori_loop(0, _CPW, body, 0)
    plsc.subcore_barrier()

    out_base = cid * _N + sid * _NPW
    pltpu.sync_copy(acc_sh.at[stripe], out_hbm.at[pl.ds(out_base, _NPW)])


def _comb_body(p_ref, na_ref, root_ref, o_ref):
    o_ref[...] = (
        p_ref[0:_N, :]
        + p_ref[_N:, :]
        + jnp.dot(na_ref[...], root_ref[...], preferred_element_type=jnp.float32)
    )


_comb_call = pl.pallas_call(
    _comb_body,
    out_shape=jax.ShapeDtypeStruct((_N, _DOUT), jnp.float32),
)


def kernel(node_attr, edge_index, edge_attr, W1, b1, W2, b2, root):
    src = edge_index[0]
    dst = edge_index[1]
    Wc = W2.reshape(_HID, _DIN, _DOUT).transpose(1, 2, 0).reshape(_DIN, _DOUT * _HID)
    b2r = b2.reshape(_DIN, _DOUT)

    x_e = _sc_gather(node_attr, src)
    msgs = _msg_call(edge_attr, x_e, W1, b1.reshape(1, _HID), Wc, b2r)
    parts = _sc_scatter(msgs, dst, jnp.zeros((_N, _DOUT), jnp.float32))
    return _comb_call(parts, node_attr, root)


# speedup vs baseline: 2.8286x; 1.8048x over previous
import functools

import jax
import jax.numpy as jnp
from jax import lax
from jax.experimental import pallas as pl
from jax.experimental.pallas import tpu as pltpu
from jax.experimental.pallas import tpu_sc as plsc

_N = 10000
_E = 160000
_DIN = 128
_DOUT = 8
_DE = 16
_HID = 128

_CHUNK = 128
_NCH = _E // _CHUNK
_NC = 2
_NS = 16
_NW = _NC * _NS
_CPW = -(-_NCH // _NW)
_NPW = _N // _NS

_sc_mesh = plsc.VectorSubcoreMesh(core_axis_name="c", subcore_axis_name="s")


@functools.partial(
    pl.kernel,
    mesh=_sc_mesh,
    out_type=jax.ShapeDtypeStruct((_E, _DIN), jnp.float32),
    scratch_types=[
        pltpu.VMEM((_CHUNK,), jnp.int32),
        pltpu.VMEM((_CHUNK, _DIN), jnp.float32),
        pltpu.SemaphoreType.DMA,
    ],
    compiler_params=pltpu.CompilerParams(use_tc_tiling_on_sc=False),
)
def _sc_gather(na_hbm, src_hbm, out_hbm, idx_v, rows_v, sem):
    wid = lax.axis_index("s") * _NC + lax.axis_index("c")

    def body(j, carry):
        c = j * _NW + wid

        @pl.when(c < _NCH)
        def _():
            base = pl.multiple_of(c * _CHUNK, _CHUNK)
            pltpu.sync_copy(src_hbm.at[pl.ds(base, _CHUNK)], idx_v)
            pltpu.async_copy(na_hbm.at[idx_v], rows_v, sem).wait()
            pltpu.sync_copy(rows_v, out_hbm.at[pl.ds(base, _CHUNK)])

        return carry

    lax.fori_loop(0, _CPW, body, 0)


_BE = 2000


def _msg_body(ea_ref, x_ref, w1_ref, b1_ref, wc_ref, b2r_ref, s_ref, o_ref):
    x = x_ref[...]
    h = jnp.maximum(
        jnp.dot(ea_ref[...], w1_ref[...], preferred_element_type=jnp.float32)
        + b1_ref[...],
        0.0,
    )
    q = jnp.dot(x, wc_ref[...], preferred_element_type=jnp.float32)
    hrep = jnp.concatenate([h] * _DOUT, axis=1)
    t = q * hrep
    m = jnp.dot(t, s_ref[...], preferred_element_type=jnp.float32)
    o_ref[...] = m + jnp.dot(x, b2r_ref[...], preferred_element_type=jnp.float32)


_msg_call = pl.pallas_call(
    _msg_body,
    grid=(_E // _BE,),
    in_specs=[
        pl.BlockSpec((_BE, _DE), lambda i: (i, 0)),
        pl.BlockSpec((_BE, _DIN), lambda i: (i, 0)),
        pl.BlockSpec((_DE, _HID), lambda i: (0, 0)),
        pl.BlockSpec((1, _HID), lambda i: (0, 0)),
        pl.BlockSpec((_DIN, _DOUT * _HID), lambda i: (0, 0)),
        pl.BlockSpec((_DIN, _DOUT), lambda i: (0, 0)),
        pl.BlockSpec((_DOUT * _HID, _DOUT), lambda i: (0, 0)),
    ],
    out_specs=pl.BlockSpec((_BE, _DOUT), lambda i: (i, 0)),
    out_shape=jax.ShapeDtypeStruct((_E, _DOUT), jnp.float32),
    compiler_params=pltpu.CompilerParams(
        dimension_semantics=("arbitrary",),
    ),
)


@functools.partial(
    pl.kernel,
    mesh=_sc_mesh,
    out_type=jax.ShapeDtypeStruct((_NC * _N, _DOUT), jnp.float32),
    scratch_types=[
        pltpu.VMEM((_CHUNK,), jnp.int32),
        pltpu.VMEM((_CHUNK, _DOUT), jnp.float32),
        pltpu.VMEM_SHARED((_N, _DOUT), jnp.float32),
    ],
    compiler_params=pltpu.CompilerParams(use_tc_tiling_on_sc=False),
)
def _sc_scatter(msg_hbm, dst_hbm, zero_hbm, out_hbm, idx_v, msg_v, acc_sh):
    cid = lax.axis_index("c")
    sid = lax.axis_index("s")
    wid = sid * _NC + cid

    stripe = pl.ds(sid * _NPW, _NPW)
    pltpu.sync_copy(zero_hbm.at[stripe], acc_sh.at[stripe])
    plsc.subcore_barrier()

    def body(j, carry):
        c = j * _NW + wid

        @pl.when(c < _NCH)
        def _():
            base = pl.multiple_of(c * _CHUNK, _CHUNK)
            pltpu.sync_copy(dst_hbm.at[pl.ds(base, _CHUNK)], idx_v)
            pltpu.sync_copy(msg_hbm.at[pl.ds(base, _CHUNK)], msg_v)
            pltpu.sync_copy(msg_v, acc_sh.at[idx_v], add=True)

        return carry

    lax.fori_loop(0, _CPW, body, 0)
    plsc.subcore_barrier()

    out_base = cid * _N + sid * _NPW
    pltpu.sync_copy(acc_sh.at[stripe], out_hbm.at[pl.ds(out_base, _NPW)])


def _comb_body(p_ref, na_ref, root_ref, o_ref):
    o_ref[...] = (
        p_ref[0:_N, :]
        + p_ref[_N:, :]
        + jnp.dot(na_ref[...], root_ref[...], preferred_element_type=jnp.float32)
    )


_comb_call = pl.pallas_call(
    _comb_body,
    out_shape=jax.ShapeDtypeStruct((_N, _DOUT), jnp.float32),
)


def kernel(node_attr, edge_index, edge_attr, W1, b1, W2, b2, root):
    src = edge_index[0]
    dst = edge_index[1]
    Wc = W2.reshape(_HID, _DIN, _DOUT).transpose(1, 2, 0).reshape(_DIN, _DOUT * _HID)
    b2r = b2.reshape(_DIN, _DOUT)
    sel = jnp.repeat(jnp.eye(_DOUT, dtype=jnp.float32), _HID, axis=0)

    x_e = _sc_gather(node_attr, src)
    msgs = _msg_call(edge_attr, x_e, W1, b1.reshape(1, _HID), Wc, b2r, sel)
    parts = _sc_scatter(msgs, dst, jnp.zeros((_N, _DOUT), jnp.float32))
    return _comb_call(parts, node_attr, root)
